# baseline (speedup 1.0000x reference)
import dataclasses
import functools

import jax
import jax.numpy as jnp
from jax import lax
from jax.experimental import pallas as pl
from jax.experimental.pallas import tpu as pltpu
from jax.experimental.pallas import tpu_sc as plsc

N = 10000
NP = 10240
E = 320000
EI = E + N
D = 128
NEG = 0.2

NTILES = 32
BLK = 128
TILE_BLKS = 88
CBLK = 8
CHUNKS = TILE_BLKS // CBLK
EP = NTILES * TILE_BLKS * BLK
DUMMY = 10200

ROWB = 2048
NROWB = NP // ROWB
SUBROWS = NP // 16


def _prep_body(x_ref, w_ref, asv_ref, adv_ref, h_ref, asrc_ref, adst_ref):
    h = jnp.dot(x_ref[...], w_ref[...], preferred_element_type=jnp.float32)
    h_ref[...] = h
    asrc_ref[...] = jnp.sum(h * asv_ref[...], axis=1).reshape(1, 1, ROWB)
    adst_ref[...] = jnp.sum(h * adv_ref[...], axis=1).reshape(1, 1, ROWB)


_prep = pl.pallas_call(
    _prep_body,
    grid=(NROWB,),
    in_specs=[
        pl.BlockSpec((ROWB, D), lambda i: (i, 0)),
        pl.BlockSpec((D, D), lambda i: (0, 0)),
        pl.BlockSpec((1, D), lambda i: (0, 0)),
        pl.BlockSpec((1, D), lambda i: (0, 0)),
    ],
    out_specs=[
        pl.BlockSpec((ROWB, D), lambda i: (i, 0)),
        pl.BlockSpec((1, 1, ROWB), lambda i: (i, 0, 0)),
        pl.BlockSpec((1, 1, ROWB), lambda i: (i, 0, 0)),
    ],
    out_shape=[
        jax.ShapeDtypeStruct((NP, D), jnp.float32),
        jax.ShapeDtypeStruct((NROWB, 1, ROWB), jnp.float32),
        jax.ShapeDtypeStruct((NROWB, 1, ROWB), jnp.float32),
    ],
)

_sc_mesh = plsc.VectorSubcoreMesh(core_axis_name="c", subcore_axis_name="s")

_sc_params = pltpu.CompilerParams()
if "needs_layout_passes" in pltpu.CompilerParams.__dataclass_fields__:
    _sc_params = dataclasses.replace(_sc_params, needs_layout_passes=False)


@functools.partial(
    pl.kernel,
    mesh=_sc_mesh,
    compiler_params=_sc_params,
    out_type=[
        jax.ShapeDtypeStruct((2, NP, D), jnp.float32),
        jax.ShapeDtypeStruct((2 * NP,), jnp.float32),
        jax.ShapeDtypeStruct((NTILES, TILE_BLKS, BLK), jnp.float32),
    ],
    scratch_types=[
        pltpu.VMEM((SUBROWS,), jnp.float32),
        pltpu.VMEM((CBLK, BLK), jnp.int32),
        pltpu.VMEM((CBLK, BLK), jnp.int32),
        pltpu.VMEM((CBLK, BLK), jnp.float32),
        pltpu.VMEM((2, BLK), jnp.float32),
        pltpu.VMEM((2, BLK), jnp.float32),
        pltpu.VMEM((BLK, D), jnp.float32),
        pltpu.VMEM((BLK, D), jnp.float32),
        pltpu.VMEM_SHARED((NP, D), jnp.float32),
        pltpu.VMEM_SHARED((NP,), jnp.float32),
        pltpu.SemaphoreType.DMA,
    ],
)
def _edge_pass(asrc_hbm, adst_hbm, src_hbm, dst_hbm, h_hbm,
               acc_hbm, den_hbm, p_hbm,
               zden_v, src_c, dst_c, p_c, as_t, ad_t, rows_v0, rows_v1,
               acc_sh, den_sh, gsem):
    cid = lax.axis_index("c")
    sid = lax.axis_index("s")
    wid = sid * 2 + cid
    zeros16 = jnp.zeros((16,), jnp.float32)

    @pl.loop(0, SUBROWS // 16)
    def _(k):
        zden_v[pl.ds(k * 16, 16)] = zeros16

    pltpu.sync_copy(zden_v, den_sh.at[pl.ds(sid * SUBROWS, SUBROWS)])

    @pl.loop(0, BLK)
    def _(r):
        for c in range(D // 16):
            rows_v0[r, pl.ds(c * 16, 16)] = zeros16

    for z in range(SUBROWS // BLK):
        pltpu.sync_copy(rows_v0, acc_sh.at[pl.ds(sid * SUBROWS + z * BLK, BLK)])
    plsc.subcore_barrier()

    rows_bufs = (rows_v0, rows_v1)

    def _issue(b):
        d0 = pltpu.async_copy(asrc_hbm.at[src_c.at[b]], as_t.at[b % 2], gsem)
        d1 = pltpu.async_copy(adst_hbm.at[dst_c.at[b]], ad_t.at[b % 2], gsem)
        d2 = pltpu.async_copy(h_hbm.at[src_c.at[b]], rows_bufs[b % 2], gsem)
        return (d0, d1, d2)

    def _process(b, descs):
        for d in descs:
            d.wait()
        rows_v = rows_bufs[b % 2]

        @pl.loop(0, BLK // 16)
        def _(g):
            a_s = as_t[b % 2, pl.ds(g * 16, 16)]
            a_d = ad_t[b % 2, pl.ds(g * 16, 16)]
            t = a_s + a_d
            e = jnp.maximum(t, NEG * t)
            m = jnp.maximum(a_d, NEG * a_d)
            p_c[b, pl.ds(g * 16, 16)] = jnp.exp(e - m)

        pltpu.sync_copy(p_c.at[b], den_sh.at[dst_c.at[b]], add=True)

        @pl.loop(0, BLK // 16)
        def _(g):
            p16 = p_c[b, pl.ds(g * 16, 16)]
            for r16 in range(16):
                r = g * 16 + r16
                pv = p16[r16]
                for c in range(D // 16):
                    rows_v[r, pl.ds(c * 16, 16)] = (
                        rows_v[r, pl.ds(c * 16, 16)] * pv)

        pltpu.sync_copy(rows_v, acc_sh.at[dst_c.at[b]], add=True)

    @pl.loop(0, CHUNKS)
    def _(ch):
        pltpu.sync_copy(src_hbm.at[wid, pl.ds(ch * CBLK, CBLK)], src_c)
        pltpu.sync_copy(dst_hbm.at[wid, pl.ds(ch * CBLK, CBLK)], dst_c)
        descs = _issue(0)
        for b in range(CBLK):
            nxt = _issue(b + 1) if b + 1 < CBLK else None
            _process(b, descs)
            descs = nxt
        pltpu.sync_copy(p_c, p_hbm.at[wid, pl.ds(ch * CBLK, CBLK)])

    plsc.subcore_barrier()

    @pl.when(sid == 0)
    def _():
        pltpu.sync_copy(den_sh, den_hbm.at[pl.ds(cid * NP, NP)])

    pltpu.sync_copy(acc_sh.at[pl.ds(sid * SUBROWS, SUBROWS)],
                    acc_hbm.at[cid, pl.ds(sid * SUBROWS, SUBROWS)])


def _fin_body(acc_ref, den_ref, bias_ref, gamma_ref, beta_ref, h_ref, inv_ref):
    num = acc_ref[0] + acc_ref[1]
    den = jnp.sum(den_ref[...], axis=(0, 1))
    inv = 1.0 / (den + 1e-16)
    out = num * inv[:, None] + bias_ref[...]
    mu = jnp.mean(out, axis=1, keepdims=True)
    var = jnp.mean((out - mu) * (out - mu), axis=1, keepdims=True)
    h_ref[...] = (out - mu) * lax.rsqrt(var + 1e-5) * gamma_ref[...] + beta_ref[...]
    inv_ref[...] = inv.reshape(1, 1, ROWB)


_finalize = pl.pallas_call(
    _fin_body,
    grid=(NROWB,),
    in_specs=[
        pl.BlockSpec((2, ROWB, D), lambda i: (0, i, 0)),
        pl.BlockSpec((2, 1, ROWB), lambda i: (0, 0, i)),
        pl.BlockSpec((1, D), lambda i: (0, 0)),
        pl.BlockSpec((1, D), lambda i: (0, 0)),
        pl.BlockSpec((1, D), lambda i: (0, 0)),
    ],
    out_specs=[
        pl.BlockSpec((ROWB, D), lambda i: (i, 0)),
        pl.BlockSpec((1, 1, ROWB), lambda i: (i, 0, 0)),
    ],
    out_shape=[
        jax.ShapeDtypeStruct((NP, D), jnp.float32),
        jax.ShapeDtypeStruct((NROWB, 1, ROWB), jnp.float32),
    ],
)


@functools.partial(
    pl.kernel,
    mesh=_sc_mesh,
    compiler_params=_sc_params,
    out_type=jax.ShapeDtypeStruct((NTILES, TILE_BLKS, BLK), jnp.float32),
    scratch_types=[
        pltpu.VMEM((NP,), jnp.float32),
        pltpu.VMEM((TILE_BLKS, BLK), jnp.int32),
        pltpu.VMEM((TILE_BLKS, BLK), jnp.float32),
        pltpu.VMEM((TILE_BLKS, BLK), jnp.float32),
    ],
)
def _alpha_pass(inv_hbm, dst_hbm, p_hbm, alpha_hbm, inv_v, dst_v, p_v, alpha_v):
    cid = lax.axis_index("c")
    sid = lax.axis_index("s")
    wid = sid * 2 + cid
    pltpu.sync_copy(inv_hbm, inv_v)
    pltpu.sync_copy(dst_hbm.at[wid], dst_v)
    pltpu.sync_copy(p_hbm.at[wid], p_v)

    @pl.loop(0, TILE_BLKS)
    def _(j):
        @pl.loop(0, BLK // 16)
        def _(g):
            didx = dst_v[j, pl.ds(g * 16, 16)]
            inv = plsc.load_gather(inv_v, [didx])
            alpha_v[j, pl.ds(g * 16, 16)] = p_v[j, pl.ds(g * 16, 16)] * inv

    pltpu.sync_copy(alpha_v, alpha_hbm.at[wid])


@jax.jit
def kernel(X, edge_index, edge_attr, W, att_src, att_dst, bias, ln_gamma, ln_beta):
    loops = jnp.arange(N, dtype=edge_index.dtype)
    ei = jnp.concatenate([edge_index, jnp.stack([loops, loops], axis=0)], axis=1)
    pad = jnp.full((EP - EI,), DUMMY, jnp.int32)
    src_pad = jnp.concatenate([ei[0], pad]).reshape(NTILES, TILE_BLKS, BLK)
    dst_pad = jnp.concatenate([ei[1], pad]).reshape(NTILES, TILE_BLKS, BLK)
    Xp = jnp.concatenate([X, jnp.zeros((NP - N, D), X.dtype)], axis=0)

    h, asrc3, adst3 = _prep(Xp, W, att_src.reshape(1, D), att_dst.reshape(1, D))
    acc, den_flat, p3 = _edge_pass(asrc3.reshape(NP), adst3.reshape(NP),
                                   src_pad, dst_pad, h)
    hnorm_p, inv3 = _finalize(acc, den_flat.reshape(2, 1, NP),
                              bias.reshape(1, D),
                              ln_gamma.reshape(1, D), ln_beta.reshape(1, D))
    alpha3 = _alpha_pass(inv3.reshape(NP), dst_pad, p3)

    H_norm = hnorm_p[:N]
    alpha = alpha3.reshape(EP)[:EI, None]
    return (H_norm, edge_index, edge_attr, ei, alpha)


# speedup vs baseline: 4.8632x; 4.8632x over previous
import dataclasses
import functools

import jax
import jax.numpy as jnp
from jax import lax
from jax.experimental import pallas as pl
from jax.experimental.pallas import tpu as pltpu
from jax.experimental.pallas import tpu_sc as plsc

N = 10000
NP = 10240
E = 320000
EI = E + N
D = 128
NEG = 0.2

NTILES = 32
BLK = 128
CBLK = 9
CHUNKS = 9
TILE_BLKS = CBLK * CHUNKS
EP = NTILES * TILE_BLKS * BLK
DUMMY = 10000

ROWB = 2048
NROWB = NP // ROWB
SUBROWS = NP // 16


def _prep_body(x_ref, w_ref, asv_ref, adv_ref, h_ref, asrc_ref, adst_ref):
    h = jnp.dot(x_ref[...], w_ref[...], preferred_element_type=jnp.float32)
    h_ref[...] = h
    asrc_ref[...] = jnp.sum(h * asv_ref[...], axis=1).reshape(1, 1, ROWB)
    adst_ref[...] = jnp.sum(h * adv_ref[...], axis=1).reshape(1, 1, ROWB)


_prep = pl.pallas_call(
    _prep_body,
    grid=(NROWB,),
    in_specs=[
        pl.BlockSpec((ROWB, D), lambda i: (i, 0)),
        pl.BlockSpec((D, D), lambda i: (0, 0)),
        pl.BlockSpec((1, D), lambda i: (0, 0)),
        pl.BlockSpec((1, D), lambda i: (0, 0)),
    ],
    out_specs=[
        pl.BlockSpec((ROWB, D), lambda i: (i, 0)),
        pl.BlockSpec((1, 1, ROWB), lambda i: (i, 0, 0)),
        pl.BlockSpec((1, 1, ROWB), lambda i: (i, 0, 0)),
    ],
    out_shape=[
        jax.ShapeDtypeStruct((NP, D), jnp.float32),
        jax.ShapeDtypeStruct((NROWB, 1, ROWB), jnp.float32),
        jax.ShapeDtypeStruct((NROWB, 1, ROWB), jnp.float32),
    ],
)

_sc_mesh = plsc.VectorSubcoreMesh(core_axis_name="c", subcore_axis_name="s")

_sc_params = pltpu.CompilerParams()
if "needs_layout_passes" in pltpu.CompilerParams.__dataclass_fields__:
    _sc_params = dataclasses.replace(_sc_params, needs_layout_passes=False)


@functools.partial(
    pl.kernel,
    mesh=_sc_mesh,
    compiler_params=_sc_params,
    out_type=[
        jax.ShapeDtypeStruct((2, NP, D), jnp.float32),
        jax.ShapeDtypeStruct((2 * NP,), jnp.float32),
        jax.ShapeDtypeStruct((NTILES, CHUNKS, CBLK, BLK), jnp.float32),
    ],
    scratch_types=[
        pltpu.VMEM((SUBROWS,), jnp.float32),
        pltpu.VMEM((CBLK, BLK), jnp.int32),
        pltpu.VMEM((CBLK, BLK), jnp.int32),
        pltpu.VMEM((CBLK, BLK), jnp.float32),
        pltpu.VMEM((2, BLK), jnp.float32),
        pltpu.VMEM((2, BLK), jnp.float32),
        pltpu.VMEM((BLK, D), jnp.float32),
        pltpu.VMEM((BLK, D), jnp.float32),
        pltpu.VMEM_SHARED((NP, D), jnp.float32),
        pltpu.VMEM_SHARED((NP,), jnp.float32),
        pltpu.SemaphoreType.DMA,
        pltpu.SemaphoreType.DMA,
    ],
)
def _edge_pass(asrc_hbm, adst_hbm, src_hbm, dst_hbm, h_hbm,
               acc_hbm, den_hbm, p_hbm,
               zden_v, src_c, dst_c, p_c, as_t, ad_t, rows_v0, rows_v1,
               acc_sh, den_sh, gsem, ssem):
    cid = lax.axis_index("c")
    sid = lax.axis_index("s")
    wid = sid * 2 + cid
    zeros16 = jnp.zeros((16,), jnp.float32)

    @pl.loop(0, SUBROWS // 16)
    def _(k):
        zden_v[pl.ds(k * 16, 16)] = zeros16

    pltpu.sync_copy(zden_v, den_sh.at[pl.ds(sid * SUBROWS, SUBROWS)])

    @pl.loop(0, BLK)
    def _(r):
        for c in range(D // 16):
            rows_v0[r, pl.ds(c * 16, 16)] = zeros16

    for z in range(SUBROWS // BLK):
        pltpu.sync_copy(rows_v0, acc_sh.at[pl.ds(sid * SUBROWS + z * BLK, BLK)])
    plsc.subcore_barrier()

    rows_bufs = (rows_v0, rows_v1)

    def _issue(b):
        d0 = pltpu.async_copy(asrc_hbm.at[src_c.at[b]], as_t.at[b % 2], gsem)
        d1 = pltpu.async_copy(adst_hbm.at[dst_c.at[b]], ad_t.at[b % 2], gsem)
        d2 = pltpu.async_copy(h_hbm.at[src_c.at[b]], rows_bufs[b % 2], gsem)
        return (d0, d1, d2)

    def _process(b, descs):
        for d in descs:
            d.wait()
        rows_v = rows_bufs[b % 2]

        @pl.loop(0, BLK // 16)
        def _(g):
            a_s = as_t[b % 2, pl.ds(g * 16, 16)]
            a_d = ad_t[b % 2, pl.ds(g * 16, 16)]
            t = a_s + a_d
            e = jnp.maximum(t, NEG * t)
            m = jnp.maximum(a_d, NEG * a_d)
            p_c[b, pl.ds(g * 16, 16)] = jnp.exp(e - m)

        pltpu.sync_copy(p_c.at[b], den_sh.at[dst_c.at[b]], add=True)

        @pl.loop(0, BLK // 16)
        def _(g):
            p16 = p_c[b, pl.ds(g * 16, 16)]
            for r16 in range(16):
                r = g * 16 + r16
                pv = p16[r16]
                for c in range(D // 16):
                    rows_v[r, pl.ds(c * 16, 16)] = (
                        rows_v[r, pl.ds(c * 16, 16)] * pv)

        return pltpu.async_copy(rows_v, acc_sh.at[dst_c.at[b]], ssem,
                                add=True)

    @pl.loop(0, CHUNKS)
    def _(ch):
        pltpu.sync_copy(src_hbm.at[wid, ch], src_c)
        pltpu.sync_copy(dst_hbm.at[wid, ch], dst_c)
        descs = _issue(0)
        sdescs = [None, None]
        for b in range(CBLK):
            if b + 1 < CBLK:
                if sdescs[(b + 1) % 2] is not None:
                    sdescs[(b + 1) % 2].wait()
                    sdescs[(b + 1) % 2] = None
                nxt = _issue(b + 1)
            else:
                nxt = None
            sdescs[b % 2] = _process(b, descs)
            descs = nxt
        for sd in sdescs:
            if sd is not None:
                sd.wait()
        pltpu.sync_copy(p_c, p_hbm.at[wid, ch])

    plsc.subcore_barrier()

    @pl.when(sid == 0)
    def _():
        pltpu.sync_copy(den_sh, den_hbm.at[pl.ds(cid * NP, NP)])

    pltpu.sync_copy(acc_sh.at[pl.ds(sid * SUBROWS, SUBROWS)],
                    acc_hbm.at[cid, pl.ds(sid * SUBROWS, SUBROWS)])


def _fin_body(acc_ref, den_ref, bias_ref, gamma_ref, beta_ref, h_ref, inv_ref):
    num = acc_ref[0] + acc_ref[1]
    den = jnp.sum(den_ref[...], axis=(0, 1))
    inv = 1.0 / (den + 1e-16)
    out = num * inv[:, None] + bias_ref[...]
    mu = jnp.mean(out, axis=1, keepdims=True)
    var = jnp.mean((out - mu) * (out - mu), axis=1, keepdims=True)
    h_ref[...] = (out - mu) * lax.rsqrt(var + 1e-5) * gamma_ref[...] + beta_ref[...]
    inv_ref[...] = inv.reshape(1, 1, ROWB)


_finalize = pl.pallas_call(
    _fin_body,
    grid=(NROWB,),
    in_specs=[
        pl.BlockSpec((2, ROWB, D), lambda i: (0, i, 0)),
        pl.BlockSpec((2, 1, ROWB), lambda i: (0, 0, i)),
        pl.BlockSpec((1, D), lambda i: (0, 0)),
        pl.BlockSpec((1, D), lambda i: (0, 0)),
        pl.BlockSpec((1, D), lambda i: (0, 0)),
    ],
    out_specs=[
        pl.BlockSpec((ROWB, D), lambda i: (i, 0)),
        pl.BlockSpec((1, 1, ROWB), lambda i: (i, 0, 0)),
    ],
    out_shape=[
        jax.ShapeDtypeStruct((NP, D), jnp.float32),
        jax.ShapeDtypeStruct((NROWB, 1, ROWB), jnp.float32),
    ],
)


@functools.partial(
    pl.kernel,
    mesh=_sc_mesh,
    compiler_params=_sc_params,
    out_type=jax.ShapeDtypeStruct((NTILES, TILE_BLKS, BLK), jnp.float32),
    scratch_types=[
        pltpu.VMEM((NP,), jnp.float32),
        pltpu.VMEM((TILE_BLKS, BLK), jnp.int32),
        pltpu.VMEM((TILE_BLKS, BLK), jnp.float32),
        pltpu.VMEM((TILE_BLKS, BLK), jnp.float32),
    ],
)
def _alpha_pass(inv_hbm, dst_hbm, p_hbm, alpha_hbm, inv_v, dst_v, p_v, alpha_v):
    cid = lax.axis_index("c")
    sid = lax.axis_index("s")
    wid = sid * 2 + cid
    pltpu.sync_copy(inv_hbm, inv_v)
    pltpu.sync_copy(dst_hbm.at[wid], dst_v)
    pltpu.sync_copy(p_hbm.at[wid], p_v)

    @pl.loop(0, TILE_BLKS)
    def _(j):
        @pl.loop(0, BLK // 16)
        def _(g):
            didx = dst_v[j, pl.ds(g * 16, 16)]
            inv = plsc.load_gather(inv_v, [didx])
            alpha_v[j, pl.ds(g * 16, 16)] = p_v[j, pl.ds(g * 16, 16)] * inv

    pltpu.sync_copy(alpha_v, alpha_hbm.at[wid])


@jax.jit
def kernel(X, edge_index, edge_attr, W, att_src, att_dst, bias, ln_gamma, ln_beta):
    loops = jnp.arange(N, dtype=edge_index.dtype)
    ei = jnp.concatenate([edge_index, jnp.stack([loops, loops], axis=0)], axis=1)
    pad = DUMMY + (jnp.arange(EP - EI, dtype=jnp.int32) % 224)
    src_pad = jnp.concatenate([ei[0], pad]).reshape(NTILES, TILE_BLKS, BLK)
    dst_pad = jnp.concatenate([ei[1], pad]).reshape(NTILES, TILE_BLKS, BLK)
    Xp = jnp.concatenate([X, jnp.zeros((NP - N, D), X.dtype)], axis=0)

    h, asrc3, adst3 = _prep(Xp, W, att_src.reshape(1, D), att_dst.reshape(1, D))
    acc, den_flat, p4 = _edge_pass(asrc3.reshape(NP), adst3.reshape(NP),
                                   src_pad.reshape(NTILES, CHUNKS, CBLK, BLK),
                                   dst_pad.reshape(NTILES, CHUNKS, CBLK, BLK),
                                   h)
    hnorm_p, inv3 = _finalize(acc, den_flat.reshape(2, 1, NP),
                              bias.reshape(1, D),
                              ln_gamma.reshape(1, D), ln_beta.reshape(1, D))
    alpha3 = _alpha_pass(inv3.reshape(NP), dst_pad,
                         p4.reshape(NTILES, TILE_BLKS, BLK))

    H_norm = hnorm_p[:N]
    alpha = alpha3.reshape(EP)[:EI, None]
    return (H_norm, edge_index, edge_attr, ei, alpha)
